# baseline (speedup 1.0000x reference)
import functools

import jax
import jax.numpy as jnp
from jax import lax
from jax.experimental import pallas as pl
from jax.experimental.pallas import tpu as pltpu
from jax.experimental.pallas import tpu_sc as plsc

N = 10000
E = 320000
D = 128
H = 16
L = 4

NC = 2
NS = 16
NPAD = 10240
NPT = NPAD // NS
EPT = E // NS
BM = 2048
CHS = 128
NCHT = 160
SLOTS = 8
LOOK = 8

_MESH = plsc.VectorSubcoreMesh(
    core_axis_name="c", subcore_axis_name="s", num_cores=NC, num_subcores=NS)


def _mm1_body(x_ref, w_ref, o_ref):
    o_ref[0] = jnp.dot(x_ref[0], w_ref[0], preferred_element_type=jnp.float32)


def _mm1(x, w):
    return pl.pallas_call(
        _mm1_body,
        grid=(L, NPAD // BM),
        in_specs=[
            pl.BlockSpec((1, BM, D), lambda i, j: (i, j, 0)),
            pl.BlockSpec((1, D, H), lambda i, j: (i, 0, 0)),
        ],
        out_specs=pl.BlockSpec((1, BM, H), lambda i, j: (i, j, 0)),
        out_shape=jax.ShapeDtypeStruct((L, NPAD, H), jnp.float32),
    )(x, w)


def _scale_body(cnt_ref, h_ref, dinv_ref, g_ref):
    deg = cnt_ref[0, 0, :] + 1.0
    dinv = jnp.broadcast_to(lax.rsqrt(deg)[:, None], (BM, H))
    dinv_ref[0] = dinv
    g_ref[0] = dinv * h_ref[0]


def _scale(cnt3, h1):
    nb = NPAD // BM
    return pl.pallas_call(
        _scale_body,
        grid=(L, nb),
        in_specs=[
            pl.BlockSpec((1, 1, BM), lambda i, j, nb=nb: (i * nb + j, 0, 0)),
            pl.BlockSpec((1, BM, H), lambda i, j: (i, j, 0)),
        ],
        out_specs=[
            pl.BlockSpec((1, BM, H), lambda i, j: (i, j, 0)),
            pl.BlockSpec((1, BM, H), lambda i, j: (i, j, 0)),
        ],
        out_shape=[
            jax.ShapeDtypeStruct((L, NPAD, H), jnp.float32),
            jax.ShapeDtypeStruct((L, NPAD, H), jnp.float32),
        ],
    )(cnt3, h1)


def _relu_body(s1_ref, h_ref, dinv_ref, b_ref, y_ref, g2_ref):
    dinv = dinv_ref[0]
    y = jnp.maximum(dinv * s1_ref[0] + dinv * dinv * h_ref[0] + b_ref[0], 0.0)
    y_ref[0] = y
    g2_ref[0] = dinv * y


def _relu_scale(s1, h1, dinvh, b1):
    return pl.pallas_call(
        _relu_body,
        grid=(L, NPAD // BM),
        in_specs=[
            pl.BlockSpec((1, BM, H), lambda i, j: (i, j, 0)),
            pl.BlockSpec((1, BM, H), lambda i, j: (i, j, 0)),
            pl.BlockSpec((1, BM, H), lambda i, j: (i, j, 0)),
            pl.BlockSpec((1, 1, H), lambda i, j: (i, 0, 0)),
        ],
        out_specs=[
            pl.BlockSpec((1, BM, H), lambda i, j: (i, j, 0)),
            pl.BlockSpec((1, BM, H), lambda i, j: (i, j, 0)),
        ],
        out_shape=[
            jax.ShapeDtypeStruct((L, NPAD, H), jnp.float32),
            jax.ShapeDtypeStruct((L, NPAD, H), jnp.float32),
        ],
    )(s1, h1, dinvh, b1)


def _final_body(s2_ref, y_ref, dinv_ref, w_ref, b_ref, o_ref):
    dinv = dinv_ref[0]
    z = dinv * s2_ref[0] + dinv * dinv * y_ref[0]
    o_ref[0] = (jnp.dot(z, w_ref[0], preferred_element_type=jnp.float32)
                + b_ref[0])


def _final(s2, y, dinvh, w2, b2):
    return pl.pallas_call(
        _final_body,
        grid=(L, NPAD // BM),
        in_specs=[
            pl.BlockSpec((1, BM, H), lambda i, j: (i, j, 0)),
            pl.BlockSpec((1, BM, H), lambda i, j: (i, j, 0)),
            pl.BlockSpec((1, BM, H), lambda i, j: (i, j, 0)),
            pl.BlockSpec((1, H, D), lambda i, j: (i, 0, 0)),
            pl.BlockSpec((1, 1, D), lambda i, j: (i, 0, 0)),
        ],
        out_specs=pl.BlockSpec((1, BM, D), lambda i, j: (i, j, 0)),
        out_shape=jax.ShapeDtypeStruct((L, NPAD, D), jnp.float32),
    )(s2, y, dinvh, w2, b2)


NHW = 2 * NPAD
MCOL = NHW // NS


@functools.partial(
    pl.kernel,
    out_type=jax.ShapeDtypeStruct((L * NPAD,), jnp.float32),
    mesh=_MESH,
    compiler_params=pltpu.CompilerParams(use_tc_tiling_on_sc=False,
                                         needs_layout_passes=False),
    scratch_types=[
        pltpu.VMEM((2 * NCHT, CHS), jnp.int32),
        pltpu.VMEM((NHW,), jnp.float32),
        pltpu.VMEM((MCOL,), jnp.float32),
        pltpu.VMEM((MCOL,), jnp.float32),
        pltpu.VMEM_SHARED((NS, NHW), jnp.float32),
    ],
)
def _sc_deg(dsta_hbm, cnt_hbm, didx_v, hist_v, tmp_v, macc_v, hist_sh):
    c = lax.axis_index("c")
    s = lax.axis_index("s")
    zeros16 = jnp.zeros((16,), jnp.float32)
    ones16 = jnp.ones((16,), jnp.float32)

    def zbody(i, _):
        hist_v[pl.ds(i * 16, 16)] = zeros16
        return 0
    lax.fori_loop(0, NHW // 16, zbody, 0)
    for mloc in range(2):
        m = c * 2 + mloc
        row0 = (m * NS + s) * NCHT
        pltpu.sync_copy(dsta_hbm.at[pl.ds(row0, NCHT)],
                        didx_v.at[pl.ds(mloc * NCHT, NCHT)])

    def cbody(ch, _):
        for k in range(CHS // 16):
            idx16 = didx_v[ch, pl.ds(k * 16, 16)]
            plsc.addupdate_scatter(hist_v, [idx16], ones16)
        return 0
    lax.fori_loop(0, 2 * NCHT, cbody, 0)
    pltpu.sync_copy(hist_v, hist_sh.at[s])
    plsc.subcore_barrier()

    def mzbody(i, _):
        macc_v[pl.ds(i * 16, 16)] = zeros16
        return 0
    lax.fori_loop(0, MCOL // 16, mzbody, 0)
    for t in range(NS):
        pltpu.sync_copy(hist_sh.at[t, pl.ds(s * MCOL, MCOL)], tmp_v)

        def abody(i, _):
            macc_v[pl.ds(i * 16, 16)] = (macc_v[pl.ds(i * 16, 16)]
                                         + tmp_v[pl.ds(i * 16, 16)])
            return 0
        lax.fori_loop(0, MCOL // 16, abody, 0)
    pltpu.sync_copy(macc_v, cnt_hbm.at[pl.ds(2 * c * NPAD + s * MCOL, MCOL)])


@functools.partial(
    pl.kernel,
    out_type=jax.ShapeDtypeStruct((L * NPAD, H), jnp.float32),
    mesh=_MESH,
    compiler_params=pltpu.CompilerParams(use_tc_tiling_on_sc=False),
    scratch_types=[
        pltpu.VMEM((NCHT, CHS), jnp.int32),
        pltpu.VMEM((NCHT, CHS), jnp.int32),
        pltpu.VMEM((SLOTS, CHS, H), jnp.float32),
        pltpu.VMEM((NPT, H), jnp.float32),
        pltpu.VMEM_SHARED((2 * NPAD, H), jnp.float32),
    ] + [pltpu.SemaphoreType.DMA] * SLOTS,
)
def _sc_scatter(g_hbm, srcg_hbm, dsta_hbm, out_hbm,
                sidx_v, didx_v, rows_v, zrows_v, acc_sh, *gsem):
    c = lax.axis_index("c")
    s = lax.axis_index("s")
    zeros16 = jnp.zeros((16,), jnp.float32)

    def zbody(i, _):
        zrows_v[i, :] = zeros16
        return 0
    lax.fori_loop(0, NPT, zbody, 0)
    for mloc in range(2):
        pltpu.sync_copy(zrows_v, acc_sh.at[pl.ds(mloc * NPAD + s * NPT, NPT)])
    plsc.subcore_barrier()

    for mloc in range(2):
        m = c * 2 + mloc
        row0 = (m * NS + s) * NCHT
        pltpu.sync_copy(srcg_hbm.at[pl.ds(row0, NCHT)], sidx_v)
        pltpu.sync_copy(dsta_hbm.at[pl.ds(row0, NCHT)], didx_v)

        def fire_g(ch, slot):
            pltpu.async_copy(g_hbm.at[sidx_v.at[ch]], rows_v.at[slot],
                             gsem[slot])

        def wait_g(ch, slot):
            pltpu.make_async_copy(g_hbm.at[sidx_v.at[ch]], rows_v.at[slot],
                                  gsem[slot]).wait()

        def sync_s(ch, slot):
            pltpu.sync_copy(rows_v.at[slot], acc_sh.at[didx_v.at[ch]],
                            add=True)

        for j in range(LOOK):
            fire_g(j, j % SLOTS)

        def obody(g2, _):
            for j in range(LOOK):
                ch = g2 * LOOK + j
                slot = j % SLOTS
                wait_g(ch, slot)
                sync_s(ch, slot)
                fire_g(ch + LOOK, slot)
            return 0
        lax.fori_loop(0, NCHT // LOOK - 1, obody, 0)
        for j in range(LOOK):
            ch = NCHT - LOOK + j
            slot = j % SLOTS
            wait_g(ch, slot)
            sync_s(ch, slot)
    plsc.subcore_barrier()
    for mloc in range(2):
        m = c * 2 + mloc
        pltpu.sync_copy(acc_sh.at[pl.ds(mloc * NPAD + s * NPT, NPT)],
                        out_hbm.at[pl.ds(m * NPAD + s * NPT, NPT)])


def kernel(inp_0, edge_index_0, W_0_1, b_0_1, W_0_2, b_0_2,
           inp_1, edge_index_1, W_1_1, b_1_1, W_1_2, b_1_2,
           inp_2, edge_index_2, W_2_1, b_2_1, W_2_2, b_2_2,
           inp_3, edge_index_3, W_3_1, b_3_1, W_3_2, b_3_2):
    xs = jnp.stack([inp_0, inp_1, inp_2, inp_3])
    xs = jnp.pad(xs, ((0, 0), (0, NPAD - N), (0, 0)))
    eis = jnp.stack([edge_index_0, edge_index_1, edge_index_2, edge_index_3])
    src3 = jnp.pad(eis[:, 0, :].reshape(L, NS, EPT),
                   ((0, 0), (0, 0), (0, NCHT * CHS - EPT)),
                   constant_values=N)
    dst3 = jnp.pad(eis[:, 1, :].reshape(L, NS, EPT),
                   ((0, 0), (0, 0), (0, NCHT * CHS - EPT)),
                   constant_values=NPAD - 8)
    srcg = (src3 + (jnp.arange(L, dtype=jnp.int32) * NPAD)[:, None, None]
            ).reshape(L * NS * NCHT, CHS)
    dsta = (dst3 + ((jnp.arange(L, dtype=jnp.int32) % 2) * NPAD)[:, None, None]
            ).reshape(L * NS * NCHT, CHS)
    w1 = jnp.stack([W_0_1, W_1_1, W_2_1, W_3_1])
    b1 = jnp.stack([b_0_1, b_1_1, b_2_1, b_3_1]).reshape(L, 1, H)
    w2 = jnp.stack([W_0_2, W_1_2, W_2_2, W_3_2])
    b2 = jnp.stack([b_0_2, b_1_2, b_2_2, b_3_2]).reshape(L, 1, D)

    h1 = _mm1(xs, w1)
    cnt = _sc_deg(dsta)
    cnt3 = cnt.reshape(L * (NPAD // BM), 1, BM)
    dinvh, g1 = _scale(cnt3, h1)
    s1 = _sc_scatter(g1.reshape(L * NPAD, H), srcg, dsta).reshape(L, NPAD, H)
    y, g2 = _relu_scale(s1, h1, dinvh, b1)
    s2 = _sc_scatter(g2.reshape(L * NPAD, H), srcg, dsta).reshape(L, NPAD, H)
    out = _final(s2, y, dinvh, w2, b2)
    return tuple(out[i, :N] for i in range(L))


# speedup vs baseline: 1.0224x; 1.0224x over previous
import functools

import jax
import jax.numpy as jnp
from jax import lax
from jax.experimental import pallas as pl
from jax.experimental.pallas import tpu as pltpu
from jax.experimental.pallas import tpu_sc as plsc

N = 10000
E = 320000
D = 128
H = 16
L = 4

NC = 2
NS = 16
NPAD = 10240
NPT = NPAD // NS
EPT = E // NS
BM = 2048
CHS = 128
NCHT = 160
SLOTS = 8
LOOK = 8

_MESH = plsc.VectorSubcoreMesh(
    core_axis_name="c", subcore_axis_name="s", num_cores=NC, num_subcores=NS)


def _mm1_body(x_ref, w_ref, o_ref):
    o_ref[0] = jnp.dot(x_ref[0], w_ref[0], preferred_element_type=jnp.float32)


def _mm1(x, w):
    return pl.pallas_call(
        _mm1_body,
        grid=(L, NPAD // BM),
        in_specs=[
            pl.BlockSpec((1, BM, D), lambda i, j: (i, j, 0)),
            pl.BlockSpec((1, D, H), lambda i, j: (i, 0, 0)),
        ],
        out_specs=pl.BlockSpec((1, BM, H), lambda i, j: (i, j, 0)),
        out_shape=jax.ShapeDtypeStruct((L, NPAD, H), jnp.float32),
    )(x, w)


def _scale_body(cnt_ref, h_ref, dinv_ref, g_ref):
    deg = cnt_ref[0, 0, :] + 1.0
    dinv = jnp.broadcast_to(lax.rsqrt(deg)[:, None], (BM, H))
    dinv_ref[0] = dinv
    g_ref[0] = dinv * h_ref[0]


def _scale(cnt3, h1):
    nb = NPAD // BM
    return pl.pallas_call(
        _scale_body,
        grid=(L, nb),
        in_specs=[
            pl.BlockSpec((1, 1, BM), lambda i, j, nb=nb: (i * nb + j, 0, 0)),
            pl.BlockSpec((1, BM, H), lambda i, j: (i, j, 0)),
        ],
        out_specs=[
            pl.BlockSpec((1, BM, H), lambda i, j: (i, j, 0)),
            pl.BlockSpec((1, BM, H), lambda i, j: (i, j, 0)),
        ],
        out_shape=[
            jax.ShapeDtypeStruct((L, NPAD, H), jnp.float32),
            jax.ShapeDtypeStruct((L, NPAD, H), jnp.float32),
        ],
    )(cnt3, h1)


def _final_body(s2_ref, y_ref, dinv_ref, w_ref, b_ref, o_ref):
    dinv = dinv_ref[0]
    z = dinv * s2_ref[0] + dinv * dinv * y_ref[0]
    o_ref[0] = (jnp.dot(z, w_ref[0], preferred_element_type=jnp.float32)
                + b_ref[0])


def _final(s2, y, dinvh, w2, b2):
    return pl.pallas_call(
        _final_body,
        grid=(L, NPAD // BM),
        in_specs=[
            pl.BlockSpec((1, BM, H), lambda i, j: (i, j, 0)),
            pl.BlockSpec((1, BM, H), lambda i, j: (i, j, 0)),
            pl.BlockSpec((1, BM, H), lambda i, j: (i, j, 0)),
            pl.BlockSpec((1, H, D), lambda i, j: (i, 0, 0)),
            pl.BlockSpec((1, 1, D), lambda i, j: (i, 0, 0)),
        ],
        out_specs=pl.BlockSpec((1, BM, D), lambda i, j: (i, j, 0)),
        out_shape=jax.ShapeDtypeStruct((L, NPAD, D), jnp.float32),
    )(s2, y, dinvh, w2, b2)


NHW = 2 * NPAD
MCOL = NHW // NS


@functools.partial(
    pl.kernel,
    out_type=jax.ShapeDtypeStruct((L * NPAD,), jnp.float32),
    mesh=_MESH,
    compiler_params=pltpu.CompilerParams(use_tc_tiling_on_sc=False,
                                         needs_layout_passes=False),
    scratch_types=[
        pltpu.VMEM((2 * NCHT, CHS), jnp.int32),
        pltpu.VMEM((NHW,), jnp.float32),
        pltpu.VMEM((MCOL,), jnp.float32),
        pltpu.VMEM((MCOL,), jnp.float32),
        pltpu.VMEM_SHARED((NS, NHW), jnp.float32),
    ],
)
def _sc_deg(dsta_hbm, cnt_hbm, didx_v, hist_v, tmp_v, macc_v, hist_sh):
    c = lax.axis_index("c")
    s = lax.axis_index("s")
    zeros16 = jnp.zeros((16,), jnp.float32)
    ones16 = jnp.ones((16,), jnp.float32)

    def zbody(i, _):
        hist_v[pl.ds(i * 16, 16)] = zeros16
        return 0
    lax.fori_loop(0, NHW // 16, zbody, 0)
    for mloc in range(2):
        m = c * 2 + mloc
        row0 = (m * NS + s) * NCHT
        pltpu.sync_copy(dsta_hbm.at[pl.ds(row0, NCHT)],
                        didx_v.at[pl.ds(mloc * NCHT, NCHT)])

    def cbody(ch, _):
        for k in range(CHS // 16):
            idx16 = didx_v[ch, pl.ds(k * 16, 16)]
            plsc.addupdate_scatter(hist_v, [idx16], ones16)
        return 0
    lax.fori_loop(0, 2 * NCHT, cbody, 0)
    pltpu.sync_copy(hist_v, hist_sh.at[s])
    plsc.subcore_barrier()

    def mzbody(i, _):
        macc_v[pl.ds(i * 16, 16)] = zeros16
        return 0
    lax.fori_loop(0, MCOL // 16, mzbody, 0)
    for t in range(NS):
        pltpu.sync_copy(hist_sh.at[t, pl.ds(s * MCOL, MCOL)], tmp_v)

        def abody(i, _):
            macc_v[pl.ds(i * 16, 16)] = (macc_v[pl.ds(i * 16, 16)]
                                         + tmp_v[pl.ds(i * 16, 16)])
            return 0
        lax.fori_loop(0, MCOL // 16, abody, 0)
    pltpu.sync_copy(macc_v, cnt_hbm.at[pl.ds(2 * c * NPAD + s * MCOL, MCOL)])


@functools.partial(
    pl.kernel,
    out_type=(jax.ShapeDtypeStruct((L * NPAD, H), jnp.float32),
              jax.ShapeDtypeStruct((L * NPAD, H), jnp.float32),
              jax.ShapeDtypeStruct((L * NPAD, H), jnp.float32)),
    mesh=_MESH,
    compiler_params=pltpu.CompilerParams(use_tc_tiling_on_sc=False,
                                         needs_layout_passes=False),
    scratch_types=[
        pltpu.VMEM((NCHT, CHS), jnp.int32),
        pltpu.VMEM((NCHT, CHS), jnp.int32),
        pltpu.VMEM((SLOTS, CHS, H), jnp.float32),
        pltpu.VMEM((NPT, H), jnp.float32),
        pltpu.VMEM((NPT, H), jnp.float32),
        pltpu.VMEM((NPT, H), jnp.float32),
        pltpu.VMEM((NPT, H), jnp.float32),
        pltpu.VMEM((H,), jnp.float32),
        pltpu.VMEM_SHARED((2 * NPAD, H), jnp.float32),
    ] + [pltpu.SemaphoreType.DMA] * SLOTS,
)
def _sc_double(g1_hbm, h1_hbm, dinv_hbm, b1_hbm, srcg_hbm,
               dsta_hbm, y_hbm, s2_hbm, g2_hbm,
               sidx_v, didx_v, rows_v, zrows_v, arow_v, brow_v, dvrow_v,
               b1_v, acc_sh, *gsem):
    c = lax.axis_index("c")
    s = lax.axis_index("s")
    zeros16 = jnp.zeros((16,), jnp.float32)

    def zbody(i, _):
        zrows_v[i, :] = zeros16
        return 0
    lax.fori_loop(0, NPT, zbody, 0)
    for mloc in range(2):
        pltpu.sync_copy(zrows_v, acc_sh.at[pl.ds(mloc * NPAD + s * NPT, NPT)])
    plsc.subcore_barrier()

    def run_pass(tbl, sidxsrc):
        for mloc in range(2):
            m = c * 2 + mloc
            row0 = (m * NS + s) * NCHT
            pltpu.sync_copy(sidxsrc.at[pl.ds(row0, NCHT)], sidx_v)
            pltpu.sync_copy(dsta_hbm.at[pl.ds(row0, NCHT)], didx_v)

            def fire_g(ch, slot):
                pltpu.async_copy(tbl.at[sidx_v.at[ch]], rows_v.at[slot],
                                 gsem[slot])

            def wait_g(ch, slot):
                pltpu.make_async_copy(tbl.at[sidx_v.at[ch]], rows_v.at[slot],
                                      gsem[slot]).wait()

            def sync_s(ch, slot):
                pltpu.sync_copy(rows_v.at[slot], acc_sh.at[didx_v.at[ch]],
                                add=True)

            for j in range(LOOK):
                fire_g(j, j % SLOTS)

            def obody(g2, _):
                for j in range(LOOK):
                    ch = g2 * LOOK + j
                    slot = j % SLOTS
                    wait_g(ch, slot)
                    sync_s(ch, slot)
                    fire_g(ch + LOOK, slot)
                return 0
            lax.fori_loop(0, NCHT // LOOK - 1, obody, 0)
            for j in range(LOOK):
                ch = NCHT - LOOK + j
                slot = j % SLOTS
                wait_g(ch, slot)
                sync_s(ch, slot)

    run_pass(g1_hbm, srcg_hbm)
    plsc.subcore_barrier()
    for mloc in range(2):
        m = c * 2 + mloc
        gbase = m * NPAD + s * NPT
        abase = mloc * NPAD + s * NPT
        pltpu.sync_copy(acc_sh.at[pl.ds(abase, NPT)], arow_v)
        pltpu.sync_copy(h1_hbm.at[pl.ds(gbase, NPT)], brow_v)
        pltpu.sync_copy(dinv_hbm.at[pl.ds(gbase, NPT)], dvrow_v)
        pltpu.sync_copy(b1_hbm.at[m], b1_v)

        def ebody(i, _):
            dv = dvrow_v[i, :]
            yv = jnp.maximum(
                dv * arow_v[i, :] + dv * dv * brow_v[i, :] + b1_v[...], 0.0)
            arow_v[i, :] = yv
            brow_v[i, :] = dv * yv
            return 0
        lax.fori_loop(0, NPT, ebody, 0)
        pltpu.sync_copy(arow_v, y_hbm.at[pl.ds(gbase, NPT)])
        pltpu.sync_copy(brow_v, g2_hbm.at[pl.ds(gbase, NPT)])
        pltpu.sync_copy(zrows_v, acc_sh.at[pl.ds(abase, NPT)])
    plsc.subcore_barrier()
    run_pass(g2_hbm, srcg_hbm)
    plsc.subcore_barrier()
    for mloc in range(2):
        m = c * 2 + mloc
        pltpu.sync_copy(acc_sh.at[pl.ds(mloc * NPAD + s * NPT, NPT)],
                        s2_hbm.at[pl.ds(m * NPAD + s * NPT, NPT)])


def kernel(inp_0, edge_index_0, W_0_1, b_0_1, W_0_2, b_0_2,
           inp_1, edge_index_1, W_1_1, b_1_1, W_1_2, b_1_2,
           inp_2, edge_index_2, W_2_1, b_2_1, W_2_2, b_2_2,
           inp_3, edge_index_3, W_3_1, b_3_1, W_3_2, b_3_2):
    xs = jnp.stack([inp_0, inp_1, inp_2, inp_3])
    xs = jnp.pad(xs, ((0, 0), (0, NPAD - N), (0, 0)))
    eis = jnp.stack([edge_index_0, edge_index_1, edge_index_2, edge_index_3])
    src3 = jnp.pad(eis[:, 0, :].reshape(L, NS, EPT),
                   ((0, 0), (0, 0), (0, NCHT * CHS - EPT)),
                   constant_values=N)
    dst3 = jnp.pad(eis[:, 1, :].reshape(L, NS, EPT),
                   ((0, 0), (0, 0), (0, NCHT * CHS - EPT)),
                   constant_values=NPAD - 8)
    srcg = (src3 + (jnp.arange(L, dtype=jnp.int32) * NPAD)[:, None, None]
            ).reshape(L * NS * NCHT, CHS)
    dsta = (dst3 + ((jnp.arange(L, dtype=jnp.int32) % 2) * NPAD)[:, None, None]
            ).reshape(L * NS * NCHT, CHS)
    w1 = jnp.stack([W_0_1, W_1_1, W_2_1, W_3_1])
    b1 = jnp.stack([b_0_1, b_1_1, b_2_1, b_3_1])
    w2 = jnp.stack([W_0_2, W_1_2, W_2_2, W_3_2])
    b2 = jnp.stack([b_0_2, b_1_2, b_2_2, b_3_2]).reshape(L, 1, D)

    h1 = _mm1(xs, w1)
    cnt = _sc_deg(dsta)
    cnt3 = cnt.reshape(L * (NPAD // BM), 1, BM)
    dinvh, g1 = _scale(cnt3, h1)
    y, s2, _ = _sc_double(g1.reshape(L * NPAD, H), h1.reshape(L * NPAD, H),
                          dinvh.reshape(L * NPAD, H), b1, srcg, dsta)
    out = _final(s2.reshape(L, NPAD, H), y.reshape(L, NPAD, H),
                 dinvh, w2, b2)
    return tuple(out[i, :N] for i in range(L))


# speedup vs baseline: 1.0438x; 1.0209x over previous
import functools

import jax
import jax.numpy as jnp
from jax import lax
from jax.experimental import pallas as pl
from jax.experimental.pallas import tpu as pltpu
from jax.experimental.pallas import tpu_sc as plsc

N = 10000
E = 320000
D = 128
H = 16
L = 4

NC = 2
NS = 16
NPAD = 10240
NPT = NPAD // NS
EPT = E // NS
BM = 2048
CHS = 128
NCHT = 160
SLOTS = 8
LOOK = 8

_MESH = plsc.VectorSubcoreMesh(
    core_axis_name="c", subcore_axis_name="s", num_cores=NC, num_subcores=NS)


def _mmscale_body(cnt_ref, x_ref, w_ref, dinv_ref, g_ref, h_ref):
    h = jnp.dot(x_ref[0], w_ref[0], preferred_element_type=jnp.float32)
    dinv = jnp.broadcast_to(
        lax.rsqrt(cnt_ref[0, 0, :] + 1.0)[:, None], (BM, H))
    h_ref[0] = h
    dinv_ref[0] = dinv
    g_ref[0] = dinv * h


def _mmscale(cnt3, x, w):
    nb = NPAD // BM
    return pl.pallas_call(
        _mmscale_body,
        grid=(L, nb),
        in_specs=[
            pl.BlockSpec((1, 1, BM), lambda i, j, nb=nb: (i * nb + j, 0, 0)),
            pl.BlockSpec((1, BM, D), lambda i, j: (i, j, 0)),
            pl.BlockSpec((1, D, H), lambda i, j: (i, 0, 0)),
        ],
        out_specs=[
            pl.BlockSpec((1, BM, H), lambda i, j: (i, j, 0)),
            pl.BlockSpec((1, BM, H), lambda i, j: (i, j, 0)),
            pl.BlockSpec((1, BM, H), lambda i, j: (i, j, 0)),
        ],
        out_shape=[
            jax.ShapeDtypeStruct((L, NPAD, H), jnp.float32),
            jax.ShapeDtypeStruct((L, NPAD, H), jnp.float32),
            jax.ShapeDtypeStruct((L, NPAD, H), jnp.float32),
        ],
    )(cnt3, x, w)


def _final_body(s2_ref, y_ref, dinv_ref, w_ref, b_ref, o_ref):
    dinv = dinv_ref[0]
    z = dinv * s2_ref[0] + dinv * dinv * y_ref[0]
    o_ref[0] = (jnp.dot(z, w_ref[0], preferred_element_type=jnp.float32)
                + b_ref[0])


def _final(s2, y, dinvh, w2, b2):
    return pl.pallas_call(
        _final_body,
        grid=(L, NPAD // BM),
        in_specs=[
            pl.BlockSpec((1, BM, H), lambda i, j: (i, j, 0)),
            pl.BlockSpec((1, BM, H), lambda i, j: (i, j, 0)),
            pl.BlockSpec((1, BM, H), lambda i, j: (i, j, 0)),
            pl.BlockSpec((1, H, D), lambda i, j: (i, 0, 0)),
            pl.BlockSpec((1, 1, D), lambda i, j: (i, 0, 0)),
        ],
        out_specs=pl.BlockSpec((1, BM, D), lambda i, j: (i, j, 0)),
        out_shape=jax.ShapeDtypeStruct((L, NPAD, D), jnp.float32),
    )(s2, y, dinvh, w2, b2)


NHW = 2 * NPAD
MCOL = NHW // NS


@functools.partial(
    pl.kernel,
    out_type=jax.ShapeDtypeStruct((L * NPAD,), jnp.float32),
    mesh=_MESH,
    compiler_params=pltpu.CompilerParams(use_tc_tiling_on_sc=False,
                                         needs_layout_passes=False),
    scratch_types=[
        pltpu.VMEM((2 * NCHT, CHS), jnp.int32),
        pltpu.VMEM((NHW,), jnp.float32),
        pltpu.VMEM((MCOL,), jnp.float32),
        pltpu.VMEM((MCOL,), jnp.float32),
        pltpu.VMEM_SHARED((NS, NHW), jnp.float32),
    ],
)
def _sc_deg(dsta_hbm, cnt_hbm, didx_v, hist_v, tmp_v, macc_v, hist_sh):
    c = lax.axis_index("c")
    s = lax.axis_index("s")
    zeros16 = jnp.zeros((16,), jnp.float32)
    ones16 = jnp.ones((16,), jnp.float32)

    def zbody(i, _):
        hist_v[pl.ds(i * 16, 16)] = zeros16
        return 0
    lax.fori_loop(0, NHW // 16, zbody, 0)
    for mloc in range(2):
        m = c * 2 + mloc
        row0 = (m * NS + s) * NCHT
        pltpu.sync_copy(dsta_hbm.at[pl.ds(row0, NCHT)],
                        didx_v.at[pl.ds(mloc * NCHT, NCHT)])

    def cbody(ch, _):
        for k in range(CHS // 16):
            idx16 = didx_v[ch, pl.ds(k * 16, 16)]
            plsc.addupdate_scatter(hist_v, [idx16], ones16)
        return 0
    lax.fori_loop(0, 2 * NCHT, cbody, 0)
    pltpu.sync_copy(hist_v, hist_sh.at[s])
    plsc.subcore_barrier()

    def mzbody(i, _):
        macc_v[pl.ds(i * 16, 16)] = zeros16
        return 0
    lax.fori_loop(0, MCOL // 16, mzbody, 0)
    for t in range(NS):
        pltpu.sync_copy(hist_sh.at[t, pl.ds(s * MCOL, MCOL)], tmp_v)

        def abody(i, _):
            macc_v[pl.ds(i * 16, 16)] = (macc_v[pl.ds(i * 16, 16)]
                                         + tmp_v[pl.ds(i * 16, 16)])
            return 0
        lax.fori_loop(0, MCOL // 16, abody, 0)
    pltpu.sync_copy(macc_v, cnt_hbm.at[pl.ds(2 * c * NPAD + s * MCOL, MCOL)])


@functools.partial(
    pl.kernel,
    out_type=(jax.ShapeDtypeStruct((L * NPAD, H), jnp.float32),
              jax.ShapeDtypeStruct((L * NPAD, H), jnp.float32),
              jax.ShapeDtypeStruct((L * NPAD, H), jnp.float32)),
    mesh=_MESH,
    compiler_params=pltpu.CompilerParams(use_tc_tiling_on_sc=False,
                                         needs_layout_passes=False),
    scratch_types=[
        pltpu.VMEM((NCHT, CHS), jnp.int32),
        pltpu.VMEM((NCHT, CHS), jnp.int32),
        pltpu.VMEM((SLOTS, CHS, H), jnp.float32),
        pltpu.VMEM((NPT, H), jnp.float32),
        pltpu.VMEM((NPT, H), jnp.float32),
        pltpu.VMEM((NPT, H), jnp.float32),
        pltpu.VMEM((NPT, H), jnp.float32),
        pltpu.VMEM((H,), jnp.float32),
        pltpu.VMEM_SHARED((2 * NPAD, H), jnp.float32),
    ] + [pltpu.SemaphoreType.DMA] * SLOTS,
)
def _sc_double(g1_hbm, h1_hbm, dinv_hbm, b1_hbm, srcg_hbm,
               dsta_hbm, y_hbm, s2_hbm, g2_hbm,
               sidx_v, didx_v, rows_v, zrows_v, arow_v, brow_v, dvrow_v,
               b1_v, acc_sh, *gsem):
    c = lax.axis_index("c")
    s = lax.axis_index("s")
    zeros16 = jnp.zeros((16,), jnp.float32)

    def zbody(i, _):
        zrows_v[i, :] = zeros16
        return 0
    lax.fori_loop(0, NPT, zbody, 0)
    for mloc in range(2):
        pltpu.sync_copy(zrows_v, acc_sh.at[pl.ds(mloc * NPAD + s * NPT, NPT)])
    plsc.subcore_barrier()

    def run_pass(tbl, sidxsrc):
        for mloc in range(2):
            m = c * 2 + mloc
            row0 = (m * NS + s) * NCHT
            pltpu.sync_copy(sidxsrc.at[pl.ds(row0, NCHT)], sidx_v)
            pltpu.sync_copy(dsta_hbm.at[pl.ds(row0, NCHT)], didx_v)

            def fire_g(ch, slot):
                pltpu.async_copy(tbl.at[sidx_v.at[ch]], rows_v.at[slot],
                                 gsem[slot])

            def wait_g(ch, slot):
                pltpu.make_async_copy(tbl.at[sidx_v.at[ch]], rows_v.at[slot],
                                      gsem[slot]).wait()

            def sync_s(ch, slot):
                pltpu.sync_copy(rows_v.at[slot], acc_sh.at[didx_v.at[ch]],
                                add=True)

            for j in range(LOOK):
                fire_g(j, j % SLOTS)

            def obody(g2, _):
                for j in range(LOOK):
                    ch = g2 * LOOK + j
                    slot = j % SLOTS
                    wait_g(ch, slot)
                    sync_s(ch, slot)
                    fire_g(ch + LOOK, slot)
                return 0
            lax.fori_loop(0, NCHT // LOOK - 1, obody, 0)
            for j in range(LOOK):
                ch = NCHT - LOOK + j
                slot = j % SLOTS
                wait_g(ch, slot)
                sync_s(ch, slot)

    run_pass(g1_hbm, srcg_hbm)
    plsc.subcore_barrier()
    for mloc in range(2):
        m = c * 2 + mloc
        gbase = m * NPAD + s * NPT
        abase = mloc * NPAD + s * NPT
        pltpu.sync_copy(acc_sh.at[pl.ds(abase, NPT)], arow_v)
        pltpu.sync_copy(h1_hbm.at[pl.ds(gbase, NPT)], brow_v)
        pltpu.sync_copy(dinv_hbm.at[pl.ds(gbase, NPT)], dvrow_v)
        pltpu.sync_copy(b1_hbm.at[m], b1_v)

        def ebody(i, _):
            dv = dvrow_v[i, :]
            yv = jnp.maximum(
                dv * arow_v[i, :] + dv * dv * brow_v[i, :] + b1_v[...], 0.0)
            arow_v[i, :] = yv
            brow_v[i, :] = dv * yv
            return 0
        lax.fori_loop(0, NPT, ebody, 0)
        pltpu.sync_copy(arow_v, y_hbm.at[pl.ds(gbase, NPT)])
        pltpu.sync_copy(brow_v, g2_hbm.at[pl.ds(gbase, NPT)])
        pltpu.sync_copy(zrows_v, acc_sh.at[pl.ds(abase, NPT)])
    plsc.subcore_barrier()
    run_pass(g2_hbm, srcg_hbm)
    plsc.subcore_barrier()
    for mloc in range(2):
        m = c * 2 + mloc
        pltpu.sync_copy(acc_sh.at[pl.ds(mloc * NPAD + s * NPT, NPT)],
                        s2_hbm.at[pl.ds(m * NPAD + s * NPT, NPT)])


def kernel(inp_0, edge_index_0, W_0_1, b_0_1, W_0_2, b_0_2,
           inp_1, edge_index_1, W_1_1, b_1_1, W_1_2, b_1_2,
           inp_2, edge_index_2, W_2_1, b_2_1, W_2_2, b_2_2,
           inp_3, edge_index_3, W_3_1, b_3_1, W_3_2, b_3_2):
    xs = jnp.stack([inp_0, inp_1, inp_2, inp_3])
    xs = jnp.pad(xs, ((0, 0), (0, NPAD - N), (0, 0)))
    eis = jnp.stack([edge_index_0, edge_index_1, edge_index_2, edge_index_3])
    src3 = jnp.pad(eis[:, 0, :].reshape(L, NS, EPT),
                   ((0, 0), (0, 0), (0, NCHT * CHS - EPT)),
                   constant_values=N)
    dst3 = jnp.pad(eis[:, 1, :].reshape(L, NS, EPT),
                   ((0, 0), (0, 0), (0, NCHT * CHS - EPT)),
                   constant_values=NPAD - 8)
    srcg = (src3 + (jnp.arange(L, dtype=jnp.int32) * NPAD)[:, None, None]
            ).reshape(L * NS * NCHT, CHS)
    dsta = (dst3 + ((jnp.arange(L, dtype=jnp.int32) % 2) * NPAD)[:, None, None]
            ).reshape(L * NS * NCHT, CHS)
    w1 = jnp.stack([W_0_1, W_1_1, W_2_1, W_3_1])
    b1 = jnp.stack([b_0_1, b_1_1, b_2_1, b_3_1])
    w2 = jnp.stack([W_0_2, W_1_2, W_2_2, W_3_2])
    b2 = jnp.stack([b_0_2, b_1_2, b_2_2, b_3_2]).reshape(L, 1, D)

    cnt = _sc_deg(dsta)
    cnt3 = cnt.reshape(L * (NPAD // BM), 1, BM)
    dinvh, g1, h1 = _mmscale(cnt3, xs, w1)
    y, s2, _ = _sc_double(g1.reshape(L * NPAD, H), h1.reshape(L * NPAD, H),
                          dinvh.reshape(L * NPAD, H), b1, srcg, dsta)
    out = _final(s2.reshape(L, NPAD, H), y.reshape(L, NPAD, H),
                 dinvh, w2, b2)
    return tuple(out[i, :N] for i in range(L))
